# baseline (speedup 1.0000x reference)
import jax
import jax.numpy as jnp
from jax.experimental import pallas as pl

SEQ_BLOCK = 256


def _make_add_kernel(batch, d_model):
    def _add_kernel(x_ref, emb_ref, out_ref):
        e = emb_ref[...]
        for b in range(batch):
            sl = pl.ds(b * d_model, d_model)
            out_ref[:, sl] = x_ref[:, sl] + e

    return _add_kernel


def kernel(x, emb_table):
    seq_len, batch, d_model = x.shape
    x2 = x.reshape(seq_len, batch * d_model)
    grid = (seq_len // SEQ_BLOCK,)
    out2 = pl.pallas_call(
        _make_add_kernel(batch, d_model),
        grid=grid,
        in_specs=[
            pl.BlockSpec((SEQ_BLOCK, batch * d_model), lambda i: (i, 0)),
            pl.BlockSpec((SEQ_BLOCK, d_model), lambda i: (i, 0)),
        ],
        out_specs=pl.BlockSpec((SEQ_BLOCK, batch * d_model), lambda i: (i, 0)),
        out_shape=jax.ShapeDtypeStruct((seq_len, batch * d_model), x.dtype),
    )(x2, emb_table[:seq_len])
    return out2.reshape(seq_len, batch, d_model)


# speedup vs baseline: 3.4427x; 3.4427x over previous
import jax
import jax.numpy as jnp
from jax.experimental import pallas as pl
from jax.experimental.pallas import tpu as pltpu

CHUNK = 128
NBUF = 4


def _make_kernel(n_chunks, batch, d_model):
    def body(x_hbm, emb_hbm, out_hbm, xbuf, ebuf, in_sems, e_sems, out_sems):
        def in_copy(i):
            slot = i % NBUF
            cx = pltpu.make_async_copy(
                x_hbm.at[pl.ds(i * CHUNK, CHUNK)], xbuf.at[slot], in_sems.at[slot])
            ce = pltpu.make_async_copy(
                emb_hbm.at[pl.ds(i * CHUNK, CHUNK)], ebuf.at[slot], e_sems.at[slot])
            cx.start()
            ce.start()
            return cx, ce

        def out_copy(i):
            slot = i % NBUF
            return pltpu.make_async_copy(
                xbuf.at[slot], out_hbm.at[pl.ds(i * CHUNK, CHUNK)], out_sems.at[slot])

        pending_in = {}
        pending_out = {}
        for i in range(min(NBUF, n_chunks)):
            pending_in[i] = in_copy(i)
        for i in range(n_chunks):
            slot = i % NBUF
            cx, ce = pending_in.pop(i)
            cx.wait()
            ce.wait()
            xbuf[slot] = xbuf[slot] + ebuf[slot][:, None, :]
            co = out_copy(i)
            co.start()
            pending_out[i] = co
            nxt = i + NBUF
            if nxt < n_chunks:
                pending_out.pop(i).wait()
                pending_in[nxt] = in_copy(nxt)
        for co in pending_out.values():
            co.wait()

    return body


def kernel(x, emb_table):
    seq_len, batch, d_model = x.shape
    n_chunks = seq_len // CHUNK
    return pl.pallas_call(
        _make_kernel(n_chunks, batch, d_model),
        in_specs=[
            pl.BlockSpec(memory_space=pl.MemorySpace.ANY),
            pl.BlockSpec(memory_space=pl.MemorySpace.ANY),
        ],
        out_specs=pl.BlockSpec(memory_space=pl.MemorySpace.ANY),
        out_shape=jax.ShapeDtypeStruct((seq_len, batch, d_model), x.dtype),
        scratch_shapes=[
            pltpu.VMEM((NBUF, CHUNK, batch, d_model), jnp.float32),
            pltpu.VMEM((NBUF, CHUNK, d_model), jnp.float32),
            pltpu.SemaphoreType.DMA((NBUF,)),
            pltpu.SemaphoreType.DMA((NBUF,)),
            pltpu.SemaphoreType.DMA((NBUF,)),
        ],
    )(x, emb_table[:seq_len])
